# baseline (speedup 1.0000x reference)
import functools

import numpy as np

import jax
import jax.numpy as jnp
from jax import lax
from jax.experimental import pallas as pl
from jax.experimental.pallas import tpu as pltpu
from jax.experimental.pallas import tpu_sc as plsc

B, S, D = 4, 8192, 768
K = 256
SB = 512
NW = 32
RPW = (B * K) // NW
CHUNK = 1024
NIDX = B * K + 8 * NW

_INT_MIN = np.int32(-(2**31))
_BITS = [np.int32(np.uint32(1 << i)) for i in range(32)]


def _copy_logits_body(x_ref, wg_ref, out_ref, lg_ref):
    x = x_ref[0]
    out_ref[0] = x
    lg = jax.lax.dot_general(
        x, wg_ref[...],
        dimension_numbers=(((1,), (0,)), ((), ())),
        preferred_element_type=jnp.float32,
    )
    lg_ref[...] = lg.reshape(1, 1, 1, SB)


def _copy_logits(inputs, Wg):
    return pl.pallas_call(
        _copy_logits_body,
        grid=(B, S // SB),
        in_specs=[
            pl.BlockSpec((1, SB, D), lambda b, j: (b, j, 0)),
            pl.BlockSpec((D, 1), lambda b, j: (0, 0)),
        ],
        out_specs=[
            pl.BlockSpec((1, SB, D), lambda b, j: (b, j, 0)),
            pl.BlockSpec((1, 1, 1, SB), lambda b, j: (b, j, 0, 0)),
        ],
        out_shape=[
            jax.ShapeDtypeStruct((B, S, D), jnp.float32),
            jax.ShapeDtypeStruct((B, S // SB, 1, SB), jnp.float32),
        ],
        compiler_params=pltpu.CompilerParams(
            dimension_semantics=("arbitrary", "arbitrary"),
        ),
    )(inputs, Wg)


def _threshold_body(lg_ref, thr_ref, off_ref):
    x = lg_ref[...]
    bits = lax.bitcast_convert_type(x, jnp.int32)
    key = bits ^ ((bits >> 31) & np.int32(0x7FFFFFFF))
    t = jnp.zeros((B, 1, 1, 1), jnp.int32)
    for bit in range(31, -1, -1):
        t2 = t | _BITS[bit]
        cmp = t2 ^ _INT_MIN
        cnt = jnp.sum((key >= cmp).astype(jnp.int32), axis=(1, 2, 3),
                      keepdims=True)
        t = jnp.where(cnt >= K, t2, t)
    tkey = t ^ _INT_MIN
    tbits = tkey ^ ((tkey >> 31) & np.int32(0x7FFFFFFF))
    tf4 = lax.bitcast_convert_type(tbits, jnp.float32)
    tf = tf4.reshape(B, 1)
    lanes = lax.broadcasted_iota(jnp.int32, (B, 16), 1)
    rows = lax.broadcasted_iota(jnp.int32, (B, 16), 0)
    thr_ref[...] = jnp.sum(jnp.where(lanes == rows, tf, 0.0), axis=0,
                           keepdims=True)
    xq = x.reshape(B, 8, 2, 1, SB)
    tf5 = tf4.reshape(B, 1, 1, 1, 1)
    gq = jnp.sum((xq > tf5).astype(jnp.int32), axis=(2, 3, 4))
    eq = jnp.sum((xq == tf5).astype(jnp.int32), axis=(2, 3, 4))
    zero = jnp.zeros((B, 1), jnp.int32)
    gcols, ecols = [zero], [zero]
    grun, erun = zero, zero
    for qq in range(7):
        grun = grun + gq[:, qq:qq + 1]
        erun = erun + eq[:, qq:qq + 1]
        gcols.append(grun)
        ecols.append(erun)
    goff = jnp.concatenate(gcols, axis=1)
    eoff = jnp.concatenate(ecols, axis=1)
    gtot = jnp.broadcast_to(jnp.sum(gq, axis=1, keepdims=True), (B, 8))
    lane3 = lax.broadcasted_iota(jnp.int32, (B, 8, 16), 2)
    g3 = lax.broadcast_in_dim(goff, (B, 8, 16), (0, 1))
    e3 = lax.broadcast_in_dim(eoff, (B, 8, 16), (0, 1))
    t3 = lax.broadcast_in_dim(gtot, (B, 8, 16), (0, 1))
    off_ref[...] = jnp.where(
        lane3 == 0, g3, jnp.where(lane3 == 1, e3,
                                  jnp.where(lane3 == 2, t3, 0)))


def _threshold(logits4):
    return pl.pallas_call(
        _threshold_body,
        out_shape=[
            jax.ShapeDtypeStruct((1, 16), jnp.float32),
            jax.ShapeDtypeStruct((B, 8, 16), jnp.int32),
        ],
    )(logits4)


_SC_MESH = plsc.VectorSubcoreMesh(core_axis_name="c", subcore_axis_name="s")


@functools.partial(
    pl.kernel,
    mesh=_SC_MESH,
    out_type=jax.ShapeDtypeStruct((NIDX,), jnp.int32),
    scratch_types=[
        pltpu.VMEM((CHUNK,), jnp.float32),
        pltpu.VMEM((16,), jnp.float32),
        pltpu.VMEM((16,), jnp.int32),
        pltpu.VMEM((2, 128), jnp.int32),
        pltpu.VMEM((2, 128), jnp.int32),
        pltpu.SemaphoreType.DMA,
    ],
    compiler_params=pltpu.CompilerParams(needs_layout_passes=False),
)
def _sc_compact(lg_hbm, thr_hbm, off_hbm, out_hbm, lgv, thrv, offv, pos2,
                val2, sem):
    c = lax.axis_index("c")
    s = lax.axis_index("s")
    b = c * 2 + s // 8
    q = s % 8
    wid = c * 16 + s
    iota16 = lax.iota(jnp.int32, 16)

    pltpu.sync_copy(lg_hbm.at[b, 2 * q, 0], lgv.at[pl.ds(0, 512)])
    pltpu.sync_copy(lg_hbm.at[b, 2 * q + 1, 0], lgv.at[pl.ds(512, 512)])
    pltpu.sync_copy(thr_hbm, thrv)
    thr = plsc.load_gather(thrv, [jnp.full((16,), 0, jnp.int32) + b])
    pltpu.sync_copy(off_hbm.at[b, q], offv)
    offrow = offv[...]
    gt_off = jnp.sum(jnp.where(iota16 == 0, offrow, 0))
    eq_off = jnp.sum(jnp.where(iota16 == 1, offrow, 0))
    gt_tot = jnp.sum(jnp.where(iota16 == 2, offrow, 0))

    dum = jnp.int32(B * K) + wid * 8 + (iota16 & 7)
    for r in range(2):
        for j in range(8):
            pos2[r, pl.ds(j * 16, 16)] = dum
    base = b * S + q * CHUNK
    obase = b * K
    kb = K - gt_tot

    def body2(i, carry):
        pg, pe, plp = carry
        v = lgv[pl.ds(i * 16, 16)]
        gvals = base + i * 16 + iota16
        mg = v > thr
        cg = plsc.cumsum(mg.astype(jnp.int32))
        tgt_g = obase + gt_off + pg + cg - 1
        lpos_g = plp + cg - 1
        plsc.store_scatter(val2, [lpos_g >> 7, lpos_g & 127], gvals, mask=mg)
        plsc.store_scatter(pos2, [lpos_g >> 7, lpos_g & 127], tgt_g, mask=mg)
        ng = jnp.sum(mg.astype(jnp.int32))
        plp = plp + ng

        me = v == thr
        ce = plsc.cumsum(me.astype(jnp.int32))
        erank = eq_off + pe + ce - 1
        ma = me & (erank < kb)
        ca = plsc.cumsum(ma.astype(jnp.int32))
        tgt_e = obase + gt_tot + erank
        lpos_e = plp + ca - 1
        plsc.store_scatter(val2, [lpos_e >> 7, lpos_e & 127], gvals, mask=ma)
        plsc.store_scatter(pos2, [lpos_e >> 7, lpos_e & 127], tgt_e, mask=ma)
        na = jnp.sum(ma.astype(jnp.int32))
        ne = jnp.sum(me.astype(jnp.int32))
        return pg + ng, pe + ne, plp + na

    lax.fori_loop(0, CHUNK // 16, body2,
                  (jnp.int32(0), jnp.int32(0), jnp.int32(0)))

    pltpu.async_copy(val2.at[0], out_hbm.at[pos2.at[0]], sem).wait()
    pltpu.async_copy(val2.at[1], out_hbm.at[pos2.at[1]], sem).wait()


@functools.partial(
    pl.kernel,
    mesh=_SC_MESH,
    out_type=jax.ShapeDtypeStruct((B * K, D), jnp.float32),
    scratch_types=[
        pltpu.VMEM((RPW,), jnp.int32),
        pltpu.VMEM((RPW, D), jnp.float32),
        pltpu.SemaphoreType.DMA,
    ],
)
def _sc_gather(table_hbm, idx_hbm, out_hbm, idx_v, rows_v, sem):
    wid = lax.axis_index("s") * 2 + lax.axis_index("c")
    pltpu.sync_copy(idx_hbm.at[pl.ds(wid * RPW, RPW)], idx_v)
    pltpu.async_copy(table_hbm.at[idx_v], rows_v, sem).wait()
    pltpu.sync_copy(rows_v, out_hbm.at[pl.ds(wid * RPW, RPW)])


def _mm_body(r_ref, w_ref, b_ref, o_ref):
    o_ref[...] = (
        jax.lax.dot_general(
            r_ref[...], w_ref[...],
            dimension_numbers=(((1,), (0,)), ((), ())),
            preferred_element_type=jnp.float32,
        )
        + b_ref[...]
    )


def _matmul(rows, Wb, bb2d):
    return pl.pallas_call(
        _mm_body,
        out_shape=jax.ShapeDtypeStruct((B * K, D), jnp.float32),
    )(rows, Wb, bb2d)


@functools.partial(
    pl.kernel,
    mesh=_SC_MESH,
    scratch_types=[
        pltpu.VMEM((RPW,), jnp.int32),
        pltpu.VMEM((RPW, D), jnp.float32),
        pltpu.SemaphoreType.DMA,
    ],
)
def _sc_scatter(idx_hbm, y_hbm, out_hbm, idx_v, rows_v, sem):
    wid = lax.axis_index("s") * 2 + lax.axis_index("c")
    pltpu.sync_copy(idx_hbm.at[pl.ds(wid * RPW, RPW)], idx_v)
    pltpu.sync_copy(y_hbm.at[pl.ds(wid * RPW, RPW)], rows_v)
    pltpu.async_copy(rows_v, out_hbm.at[idx_v], sem).wait()


def kernel(inputs, Wg, bg, Wb, bb):
    del bg
    out0, logits4 = _copy_logits(inputs, Wg)
    thr, offs = _threshold(logits4)
    gidx = _sc_compact(logits4, thr.reshape(16), offs)
    rows = _sc_gather(inputs.reshape(B * S, D), gidx)
    y = _matmul(rows, Wb, bb.reshape(1, D))
    out_ref = jax.new_ref(out0.reshape(B * S, D))
    _sc_scatter(gidx, y, out_ref)
    return out_ref[...].reshape(B, S, D)


# speedup vs baseline: 3.5965x; 3.5965x over previous
import functools

import numpy as np

import jax
import jax.numpy as jnp
from jax import lax
from jax.experimental import pallas as pl
from jax.experimental.pallas import tpu as pltpu
from jax.experimental.pallas import tpu_sc as plsc

B, S, D = 4, 8192, 768
K = 256
SB = 512
NW = 32
RPW = (B * K) // NW
CHUNK = 1024
NIDX = B * K + 256 * NW

_INT_MIN = np.int32(-(2**31))
_BITS = [np.int32(np.uint32(1 << i)) for i in range(32)]


def _copy_logits_body(x_ref, wg_ref, out_ref, lg_ref):
    x = x_ref[0]
    out_ref[0] = x
    lg = jax.lax.dot_general(
        x, wg_ref[...],
        dimension_numbers=(((1,), (0,)), ((), ())),
        preferred_element_type=jnp.float32,
    )
    lg_ref[...] = lg.reshape(1, 1, 1, SB)


def _copy_logits(inputs, Wg):
    return pl.pallas_call(
        _copy_logits_body,
        grid=(B, S // SB),
        in_specs=[
            pl.BlockSpec((1, SB, D), lambda b, j: (b, j, 0)),
            pl.BlockSpec((D, 1), lambda b, j: (0, 0)),
        ],
        out_specs=[
            pl.BlockSpec((1, SB, D), lambda b, j: (b, j, 0)),
            pl.BlockSpec((1, 1, 1, SB), lambda b, j: (b, j, 0, 0)),
        ],
        out_shape=[
            jax.ShapeDtypeStruct((B, S, D), jnp.float32),
            jax.ShapeDtypeStruct((B, S // SB, 1, SB), jnp.float32),
        ],
        compiler_params=pltpu.CompilerParams(
            dimension_semantics=("arbitrary", "arbitrary"),
        ),
    )(inputs, Wg)


def _threshold_body(lg_ref, thr_ref, off_ref):
    x = lg_ref[...]
    bits = lax.bitcast_convert_type(x, jnp.int32)
    key = bits ^ ((bits >> 31) & np.int32(0x7FFFFFFF))
    t = jnp.zeros((B, 1, 1, 1), jnp.int32)
    for bit in range(31, -1, -1):
        t2 = t | _BITS[bit]
        cmp = t2 ^ _INT_MIN
        cnt = jnp.sum((key >= cmp).astype(jnp.int32), axis=(1, 2, 3),
                      keepdims=True)
        t = jnp.where(cnt >= K, t2, t)
    tkey = t ^ _INT_MIN
    tbits = tkey ^ ((tkey >> 31) & np.int32(0x7FFFFFFF))
    tf4 = lax.bitcast_convert_type(tbits, jnp.float32)
    tf = tf4.reshape(B, 1)
    lanes = lax.broadcasted_iota(jnp.int32, (B, 16), 1)
    rows = lax.broadcasted_iota(jnp.int32, (B, 16), 0)
    thr_ref[...] = jnp.sum(jnp.where(lanes == rows, tf, 0.0), axis=0,
                           keepdims=True)
    xq = x.reshape(B, 8, 2, 1, SB)
    tf5 = tf4.reshape(B, 1, 1, 1, 1)
    gq = jnp.sum((xq > tf5).astype(jnp.int32), axis=(2, 3, 4))
    eq = jnp.sum((xq == tf5).astype(jnp.int32), axis=(2, 3, 4))
    zero = jnp.zeros((B, 1), jnp.int32)
    gcols, ecols = [zero], [zero]
    grun, erun = zero, zero
    for qq in range(7):
        grun = grun + gq[:, qq:qq + 1]
        erun = erun + eq[:, qq:qq + 1]
        gcols.append(grun)
        ecols.append(erun)
    goff = jnp.concatenate(gcols, axis=1)
    eoff = jnp.concatenate(ecols, axis=1)
    gtot = jnp.broadcast_to(jnp.sum(gq, axis=1, keepdims=True), (B, 8))
    lane3 = lax.broadcasted_iota(jnp.int32, (B, 8, 16), 2)
    g3 = lax.broadcast_in_dim(goff, (B, 8, 16), (0, 1))
    e3 = lax.broadcast_in_dim(eoff, (B, 8, 16), (0, 1))
    t3 = lax.broadcast_in_dim(gtot, (B, 8, 16), (0, 1))
    off_ref[...] = jnp.where(
        lane3 == 0, g3, jnp.where(lane3 == 1, e3,
                                  jnp.where(lane3 == 2, t3, 0)))


def _threshold(logits4):
    return pl.pallas_call(
        _threshold_body,
        out_shape=[
            jax.ShapeDtypeStruct((1, 16), jnp.float32),
            jax.ShapeDtypeStruct((B, 8, 16), jnp.int32),
        ],
    )(logits4)


_SC_MESH = plsc.VectorSubcoreMesh(core_axis_name="c", subcore_axis_name="s")


@functools.partial(
    pl.kernel,
    mesh=_SC_MESH,
    out_type=jax.ShapeDtypeStruct((NIDX,), jnp.int32),
    scratch_types=[
        pltpu.VMEM((CHUNK,), jnp.float32),
        pltpu.VMEM((16,), jnp.float32),
        pltpu.VMEM((16,), jnp.int32),
        pltpu.VMEM((2, 128), jnp.int32),
        pltpu.VMEM((2, 128), jnp.int32),
        pltpu.SemaphoreType.DMA,
    ],
    compiler_params=pltpu.CompilerParams(needs_layout_passes=False),
)
def _sc_compact(lg_hbm, thr_hbm, off_hbm, out_hbm, lgv, thrv, offv, pos2,
                val2, sem):
    c = lax.axis_index("c")
    s = lax.axis_index("s")
    b = c * 2 + s // 8
    q = s % 8
    wid = c * 16 + s
    iota16 = lax.iota(jnp.int32, 16)

    pltpu.sync_copy(lg_hbm.at[b, 2 * q, 0], lgv.at[pl.ds(0, 512)])
    pltpu.sync_copy(lg_hbm.at[b, 2 * q + 1, 0], lgv.at[pl.ds(512, 512)])
    pltpu.sync_copy(thr_hbm, thrv)
    thr = plsc.load_gather(thrv, [jnp.full((16,), 0, jnp.int32) + b])
    pltpu.sync_copy(off_hbm.at[b, q], offv)
    offrow = offv[...]
    gt_off = jnp.sum(jnp.where(iota16 == 0, offrow, 0))
    eq_off = jnp.sum(jnp.where(iota16 == 1, offrow, 0))
    gt_tot = jnp.sum(jnp.where(iota16 == 2, offrow, 0))

    dbase = jnp.int32(B * K) + wid * 256
    for r in range(2):
        for j in range(8):
            pos2[r, pl.ds(j * 16, 16)] = dbase + r * 128 + j * 16 + iota16
    base = b * S + q * CHUNK
    obase = b * K
    kb = K - gt_tot

    def body2(i, carry):
        pg, pe, plp = carry
        v = lgv[pl.ds(i * 16, 16)]
        gvals = base + i * 16 + iota16
        mg = v > thr
        cg = plsc.cumsum(mg.astype(jnp.int32))
        tgt_g = obase + gt_off + pg + cg - 1
        lpos_g = plp + cg - 1
        plsc.store_scatter(val2, [lpos_g >> 7, lpos_g & 127], gvals, mask=mg)
        plsc.store_scatter(pos2, [lpos_g >> 7, lpos_g & 127], tgt_g, mask=mg)
        ng = jnp.sum(mg.astype(jnp.int32))
        plp = plp + ng

        me = v == thr
        ce = plsc.cumsum(me.astype(jnp.int32))
        erank = eq_off + pe + ce - 1
        ma = me & (erank < kb)
        ca = plsc.cumsum(ma.astype(jnp.int32))
        tgt_e = obase + gt_tot + erank
        lpos_e = plp + ca - 1
        plsc.store_scatter(val2, [lpos_e >> 7, lpos_e & 127], gvals, mask=ma)
        plsc.store_scatter(pos2, [lpos_e >> 7, lpos_e & 127], tgt_e, mask=ma)
        na = jnp.sum(ma.astype(jnp.int32))
        ne = jnp.sum(me.astype(jnp.int32))
        return pg + ng, pe + ne, plp + na

    lax.fori_loop(0, CHUNK // 16, body2,
                  (jnp.int32(0), jnp.int32(0), jnp.int32(0)))

    pltpu.async_copy(val2.at[0], out_hbm.at[pos2.at[0]], sem).wait()
    pltpu.async_copy(val2.at[1], out_hbm.at[pos2.at[1]], sem).wait()


@functools.partial(
    pl.kernel,
    mesh=_SC_MESH,
    out_type=jax.ShapeDtypeStruct((B * K, D), jnp.float32),
    scratch_types=[
        pltpu.VMEM((RPW,), jnp.int32),
        pltpu.VMEM((RPW, D), jnp.float32),
        pltpu.SemaphoreType.DMA,
    ],
)
def _sc_gather(table_hbm, idx_hbm, out_hbm, idx_v, rows_v, sem):
    wid = lax.axis_index("s") * 2 + lax.axis_index("c")
    pltpu.sync_copy(idx_hbm.at[pl.ds(wid * RPW, RPW)], idx_v)
    pltpu.async_copy(table_hbm.at[idx_v], rows_v, sem).wait()
    pltpu.sync_copy(rows_v, out_hbm.at[pl.ds(wid * RPW, RPW)])


def _mm_body(r_ref, w_ref, b_ref, o_ref):
    o_ref[...] = (
        jax.lax.dot_general(
            r_ref[...], w_ref[...],
            dimension_numbers=(((1,), (0,)), ((), ())),
            preferred_element_type=jnp.float32,
        )
        + b_ref[...]
    )


def _matmul(rows, Wb, bb2d):
    return pl.pallas_call(
        _mm_body,
        out_shape=jax.ShapeDtypeStruct((B * K, D), jnp.float32),
    )(rows, Wb, bb2d)


@functools.partial(
    pl.kernel,
    mesh=_SC_MESH,
    scratch_types=[
        pltpu.VMEM((RPW,), jnp.int32),
        pltpu.VMEM((RPW, D), jnp.float32),
        pltpu.SemaphoreType.DMA,
    ],
)
def _sc_scatter(idx_hbm, y_hbm, out_hbm, idx_v, rows_v, sem):
    wid = lax.axis_index("s") * 2 + lax.axis_index("c")
    pltpu.sync_copy(idx_hbm.at[pl.ds(wid * RPW, RPW)], idx_v)
    pltpu.sync_copy(y_hbm.at[pl.ds(wid * RPW, RPW)], rows_v)
    pltpu.async_copy(rows_v, out_hbm.at[idx_v], sem).wait()


def kernel(inputs, Wg, bg, Wb, bb):
    del bg
    out0, logits4 = _copy_logits(inputs, Wg)
    thr, offs = _threshold(logits4)
    gidx = _sc_compact(logits4, thr.reshape(16), offs)
    rows = _sc_gather(inputs.reshape(B * S, D), gidx)
    y = _matmul(rows, Wb, bb.reshape(1, D))
    out_ref = jax.new_ref(out0.reshape(B * S, D))
    _sc_scatter(gidx, y, out_ref)
    return out_ref[...].reshape(B, S, D)
